# baseline (speedup 1.0000x reference)
import functools

import jax
import jax.numpy as jnp
from jax import lax
from jax.experimental import pallas as pl
from jax.experimental.pallas import tpu as pltpu
from jax.experimental.pallas import tpu_sc as plsc

_EPS = 1e-5
_TILE = 4096
_TR = 16


def _sc_gather(table, idx):
  info = plsc.get_sparse_core_info()
  nw = info.num_cores * info.num_subcores
  r, d = idx.shape[0], table.shape[1]
  b_per_w = r // nw
  ch = 2048
  nch = b_per_w // ch
  mesh = plsc.VectorSubcoreMesh(core_axis_name="c", subcore_axis_name="s")

  v = table.shape[0]

  @functools.partial(
      pl.kernel, mesh=mesh,
      compiler_params=pltpu.CompilerParams(needs_layout_passes=False),
      out_type=jax.ShapeDtypeStruct((r * d,), jnp.float32),
      scratch_types=[
          pltpu.VMEM((v * d,), jnp.float32),
          pltpu.VMEM((ch,), jnp.int32),
          pltpu.VMEM((ch * d,), jnp.float32),
      ],
  )
  def k(table_hbm, idx_hbm, out_hbm, table_v, idx_v, out_v):
    wid = lax.axis_index("s") * info.num_cores + lax.axis_index("c")
    base = wid * b_per_w
    pltpu.sync_copy(table_hbm, table_v)
    io32 = lax.iota(jnp.int32, 16) * d

    def chunk(i, carry):
      off = base + i * ch
      pltpu.sync_copy(idx_hbm.at[pl.ds(off, ch)], idx_v)

      @plsc.parallel_loop(0, ch // 16, unroll=4)
      def group(g):
        iv = idx_v[pl.ds(g * 16, 16)] * d
        po = g * (16 * d) + io32
        for c in range(d):
          piece = plsc.load_gather(table_v, [iv + c])
          plsc.store_scatter(out_v, [po + c], piece)
      pltpu.sync_copy(out_v, out_hbm.at[pl.ds(off * d, ch * d)])
      return carry

    lax.fori_loop(0, nch, chunk, 0)

  return k(table.reshape(v * d), idx).reshape(r, d)


def _stq(h):
  return jnp.concatenate(
      [jnp.sum(h, axis=0, keepdims=True),
       jnp.sum(h * h, axis=0, keepdims=True)], axis=0)


def _affine(st, g, b, m):
  mean = st[0:1, :] * (1.0 / m)
  var = st[1:2, :] * (1.0 / m) - mean * mean
  alpha = g / jnp.sqrt(var + _EPS)
  beta = b - mean * alpha
  return alpha, beta


def _p1_body(xd_ref, xa_ref, xc_ref, wd_ref, wa_ref, wc_ref,
             od_ref, oa_ref, oc_ref):
  pid = pl.program_id(0)
  hd = jnp.dot(xd_ref[...], wd_ref[...], preferred_element_type=jnp.float32)
  ha = jnp.dot(xa_ref[...], wa_ref[...], preferred_element_type=jnp.float32)
  hc = xc_ref[...] * wc_ref[...]
  sd, sa, sc = _stq(hd), _stq(ha), _stq(hc)

  @pl.when(pid == 0)
  def _():
    od_ref[...] = sd
    oa_ref[...] = sa
    oc_ref[...] = sc

  @pl.when(pid != 0)
  def _():
    od_ref[...] += sd
    oa_ref[...] += sa
    oc_ref[...] += sc


def _pass1(xd, xa, xc, wd1, wa1, wc1):
  r = xd.shape[0]
  grid = (r // _TILE,)
  full = lambda s: pl.BlockSpec(s, lambda i: (0,) * len(s))
  st = jax.ShapeDtypeStruct((2, wd1.shape[1]), jnp.float32)
  return pl.pallas_call(
      _p1_body,
      grid=grid,
      in_specs=[
          pl.BlockSpec((_TILE, 64), lambda i: (i, 0)),
          pl.BlockSpec((_TILE, 64), lambda i: (i, 0)),
          pl.BlockSpec((_TILE, 1), lambda i: (i, 0)),
          full(wd1.shape), full(wa1.shape), full(wc1.shape),
      ],
      out_specs=[full((2, wd1.shape[1]))] * 3,
      out_shape=[st, st, st],
      compiler_params=pltpu.CompilerParams(
          dimension_semantics=("arbitrary",)),
  )(xd, xa, xc, wd1, wa1, wc1)


def _a1(x, w1, al, be):
  h1 = jnp.dot(x, w1, preferred_element_type=jnp.float32)
  return jnp.maximum(h1 * al + be, 0.0)


def _p2_body(m, xd_ref, xa_ref, xc_ref, s1d_ref, s1a_ref, s1c_ref,
             wd1_ref, gd1_ref, bd1_ref, wd2_ref,
             wa1_ref, ga1_ref, ba1_ref, wa2_ref,
             wc1_ref, gc1_ref, bc1_ref, wc2_ref,
             od_ref, oa_ref, oc_ref):
  pid = pl.program_id(0)

  def branch(x_ref, s1_ref, w1_ref, g1_ref, b1_ref, w2_ref, is_cd):
    al, be = _affine(s1_ref[...], g1_ref[...], b1_ref[...], m)
    if is_cd:
      a1 = jnp.maximum(x_ref[...] * w1_ref[...] * al + be, 0.0)
    else:
      a1 = _a1(x_ref[...], w1_ref[...], al, be)
    h2 = jnp.dot(a1, w2_ref[...], preferred_element_type=jnp.float32)
    return _stq(h2)

  sd = branch(xd_ref, s1d_ref, wd1_ref, gd1_ref, bd1_ref, wd2_ref, False)
  sa = branch(xa_ref, s1a_ref, wa1_ref, ga1_ref, ba1_ref, wa2_ref, False)
  sc = branch(xc_ref, s1c_ref, wc1_ref, gc1_ref, bc1_ref, wc2_ref, True)

  @pl.when(pid == 0)
  def _():
    od_ref[...] = sd
    oa_ref[...] = sa
    oc_ref[...] = sc

  @pl.when(pid != 0)
  def _():
    od_ref[...] += sd
    oa_ref[...] += sa
    oc_ref[...] += sc


def _pass2(xd, xa, xc, s1d, s1a, s1c,
           wd1, gd1, bd1, wd2, wa1, ga1, ba1, wa2, wc1, gc1, bc1, wc2):
  r = xd.shape[0]
  h = wd1.shape[1]
  grid = (r // _TILE,)
  full = lambda s: pl.BlockSpec(s, lambda i: (0,) * len(s))
  st = jax.ShapeDtypeStruct((2, h), jnp.float32)
  smalls = [s1d, s1a, s1c, wd1, gd1, bd1, wd2, wa1, ga1, ba1, wa2,
            wc1, gc1, bc1, wc2]
  return pl.pallas_call(
      functools.partial(_p2_body, float(r)),
      grid=grid,
      in_specs=[
          pl.BlockSpec((_TILE, 64), lambda i: (i, 0)),
          pl.BlockSpec((_TILE, 64), lambda i: (i, 0)),
          pl.BlockSpec((_TILE, 1), lambda i: (i, 0)),
      ] + [full(x.shape) for x in smalls],
      out_specs=[full((2, h))] * 3,
      out_shape=[st, st, st],
      compiler_params=pltpu.CompilerParams(
          dimension_semantics=("arbitrary",)),
  )(xd, xa, xc, *smalls)


def _p3_body(m, n, nh,
             xd_ref, xa_ref, xc_ref, sp_ref, spat_ref, ain_ref,
             acol_ref, arow_ref,
             s1d_ref, s2d_ref, s1a_ref, s2a_ref, s1c_ref, s2c_ref,
             wd1_ref, gd1_ref, bd1_ref, wd2_ref, gd2_ref, bd2_ref,
             wa1_ref, ga1_ref, ba1_ref, wa2_ref, ga2_ref, ba2_ref,
             wc1_ref, gc1_ref, bc1_ref, wc2_ref, gc2_ref, bc2_ref,
             tvd_ref, gam_ref, out_ref):
  rt = pl.program_id(1)

  def branch(x_ref, s1_ref, s2_ref, w1_ref, g1_ref, b1_ref,
             w2_ref, g2_ref, b2_ref, is_cd):
    al1, be1 = _affine(s1_ref[...], g1_ref[...], b1_ref[...], m)
    al2, be2 = _affine(s2_ref[...], g2_ref[...], b2_ref[...], m)
    if is_cd:
      a1 = jnp.maximum(x_ref[...] * w1_ref[...] * al1 + be1, 0.0)
    else:
      a1 = _a1(x_ref[...], w1_ref[...], al1, be1)
    h2 = jnp.dot(a1, w2_ref[...], preferred_element_type=jnp.float32)
    return jnp.maximum(h2 * al2 + be2, 0.0)

  s = branch(xd_ref, s1d_ref, s2d_ref, wd1_ref, gd1_ref, bd1_ref,
             wd2_ref, gd2_ref, bd2_ref, False)
  s += branch(xa_ref, s1a_ref, s2a_ref, wa1_ref, ga1_ref, ba1_ref,
              wa2_ref, ga2_ref, ba2_ref, False)
  s += branch(xc_ref, s1c_ref, s2c_ref, wc1_ref, gc1_ref, bc1_ref,
              wc2_ref, gc2_ref, bc2_ref, True)
  s += sp_ref[...]

  st = jnp.transpose(s, (1, 0)).reshape(nh, _TR, n)
  mask = (spat_ref[0] > 1)[None, :, :]
  ain = ain_ref[0][None, :, :]
  tot = ain + st
  gam = gam_ref[0, 0]
  inner = ain + jnp.where(mask, tot * gam, tot)

  tvec = tvd_ref[0, :]
  acol = acol_ref[0, pl.ds(1 + rt * _TR, _TR), 0]
  border = 2.0 * acol[None, :] + tvec[:, None]
  chunk = jnp.concatenate([border[:, :, None], inner], axis=2)
  out_ref[0, :, pl.ds(8 + rt * _TR, _TR), :] = chunk

  @pl.when(rt == 0)
  def _():
    row0 = arow_ref[0, 0, :]
    out_ref[0, :, 7, :] = 2.0 * row0[None, :] + tvec[:, None]


def _pass3(xd, xa, xc, sp_rows, spat, a_in, a_col, a_row0,
           stats, weights, tvd, gam):
  b, n, _ = spat.shape
  nh = tvd.shape[1]
  r = xd.shape[0]
  nt = n // _TR
  full = lambda s: pl.BlockSpec(s, lambda i, j: (0,) * len(s))
  rows = lambda w: pl.BlockSpec((_TILE, w), lambda i, j: (i * nt + j, 0))
  smalls = list(stats) + list(weights) + [tvd, gam]
  return pl.pallas_call(
      functools.partial(_p3_body, float(r), n, nh),
      grid=(b, nt),
      in_specs=[
          rows(64), rows(64), rows(1), rows(nh),
          pl.BlockSpec((1, _TR, n), lambda i, j: (i, j, 0)),
          pl.BlockSpec((1, _TR, n), lambda i, j: (i, j, 0)),
          pl.BlockSpec((1, n + 1, 1), lambda i, j: (i, 0, 0)),
          pl.BlockSpec((1, 1, n + 1), lambda i, j: (i, 0, 0)),
      ] + [full(x.shape) for x in smalls],
      out_specs=pl.BlockSpec((1, nh, n + 8, n + 1),
                             lambda i, j: (i, 0, 0, 0)),
      out_shape=jax.ShapeDtypeStruct((b, nh, n + 8, n + 1), jnp.float32),
      compiler_params=pltpu.CompilerParams(
          dimension_semantics=("arbitrary", "arbitrary")),
  )(xd, xa, xc, sp_rows, spat, a_in, a_col, a_row0, *smalls)


def kernel(attn_bias, spatial_pos, d2_distance, ang_distance,
           centroid_distance, edge_data, edge_type, edge_len, edge_ang,
           edge_conv, edge_path, edge_padding_mask, graph, node_feat,
           spatial_pos_W, gt_vd, gamma, d2_W1, d2_g1, d2_b1, d2_W2,
           d2_g2, d2_b2, ang_W1, ang_g1, ang_b1, ang_W2, ang_g2, ang_b2,
           cd_W1, cd_g1, cd_b1, cd_W2, cd_g2, cd_b2):
  b, n, _ = spatial_pos.shape
  h = spatial_pos_W.shape[1]
  r = b * n * n

  xd = d2_distance.reshape(r, 64)
  xa = ang_distance.reshape(r, 64)
  xc = centroid_distance.reshape(r, 1)
  idx = spatial_pos.reshape(r).astype(jnp.int32)

  sp_rows = _sc_gather(spatial_pos_W, idx)

  rs = lambda v: v.reshape(1, h)
  s1d, s1a, s1c = _pass1(xd, xa, xc, d2_W1, ang_W1, cd_W1)
  s2d, s2a, s2c = _pass2(
      xd, xa, xc, s1d, s1a, s1c,
      d2_W1, rs(d2_g1), rs(d2_b1), d2_W2,
      ang_W1, rs(ang_g1), rs(ang_b1), ang_W2,
      cd_W1, rs(cd_g1), rs(cd_b1), cd_W2)

  a_in = attn_bias[:, 1:, 1:]
  a_col = attn_bias[:, :, 0].reshape(b, n + 1, 1)
  a_row0 = attn_bias[:, 0, :].reshape(b, 1, n + 1)

  stats = (s1d, s2d, s1a, s2a, s1c, s2c)
  weights = (d2_W1, rs(d2_g1), rs(d2_b1), d2_W2, rs(d2_g2), rs(d2_b2),
             ang_W1, rs(ang_g1), rs(ang_b1), ang_W2, rs(ang_g2), rs(ang_b2),
             cd_W1, rs(cd_g1), rs(cd_b1), cd_W2, rs(cd_g2), rs(cd_b2))
  out_pad = _pass3(xd, xa, xc, sp_rows, spatial_pos, a_in, a_col, a_row0,
                   stats, weights, gt_vd, gamma.reshape(1, 1))
  return out_pad[:, :, 7:7 + n + 1, :]


# speedup vs baseline: 1.0544x; 1.0544x over previous
import functools

import jax
import jax.numpy as jnp
from jax import lax
from jax.experimental import pallas as pl
from jax.experimental.pallas import tpu as pltpu
from jax.experimental.pallas import tpu_sc as plsc

_EPS = 1e-5
_TILE = 4096
_TR = 16


def _sc_gather(table, idx):
  info = plsc.get_sparse_core_info()
  nw = info.num_cores * info.num_subcores
  r, d = idx.shape[0], table.shape[1]
  b_per_w = r // nw
  ch = 2048
  nch = b_per_w // ch
  mesh = plsc.VectorSubcoreMesh(core_axis_name="c", subcore_axis_name="s")

  v = table.shape[0]

  @functools.partial(
      pl.kernel, mesh=mesh,
      compiler_params=pltpu.CompilerParams(needs_layout_passes=False),
      out_type=jax.ShapeDtypeStruct((r * d,), jnp.float32),
      scratch_types=[
          pltpu.VMEM((v * d,), jnp.float32),
          pltpu.VMEM((ch,), jnp.int32),
          pltpu.VMEM((ch * d,), jnp.float32),
      ],
  )
  def k(table_hbm, idx_hbm, out_hbm, table_v, idx_v, out_v):
    wid = lax.axis_index("s") * info.num_cores + lax.axis_index("c")
    base = wid * b_per_w
    pltpu.sync_copy(table_hbm, table_v)
    io32 = lax.iota(jnp.int32, 16) * d

    def chunk(i, carry):
      off = base + i * ch
      pltpu.sync_copy(idx_hbm.at[pl.ds(off, ch)], idx_v)

      @plsc.parallel_loop(0, ch // 16, unroll=4)
      def group(g):
        iv = idx_v[pl.ds(g * 16, 16)] * d
        po = g * (16 * d) + io32
        for c in range(d):
          piece = plsc.load_gather(table_v, [iv + c])
          plsc.store_scatter(out_v, [po + c], piece)
      pltpu.sync_copy(out_v, out_hbm.at[pl.ds(off * d, ch * d)])
      return carry

    lax.fori_loop(0, nch, chunk, 0)

  return k(table.reshape(v * d), idx).reshape(r, d)


def _stq(h):
  return jnp.concatenate(
      [jnp.sum(h, axis=0, keepdims=True),
       jnp.sum(h * h, axis=0, keepdims=True)], axis=0)


def _affine(st, g, b, m):
  mean = st[0:1, :] * (1.0 / m)
  var = st[1:2, :] * (1.0 / m) - mean * mean
  alpha = g / jnp.sqrt(var + _EPS)
  beta = b - mean * alpha
  return alpha, beta


def _p1_body(xd_ref, xa_ref, xc_ref, wd_ref, wa_ref, wc_ref,
             od_ref, oa_ref, oc_ref):
  pid = pl.program_id(0)
  hd = jnp.dot(xd_ref[...], wd_ref[...], preferred_element_type=jnp.float32)
  ha = jnp.dot(xa_ref[...], wa_ref[...], preferred_element_type=jnp.float32)
  hc = xc_ref[...] * wc_ref[...]
  sd, sa, sc = _stq(hd), _stq(ha), _stq(hc)

  @pl.when(pid == 0)
  def _():
    od_ref[...] = sd
    oa_ref[...] = sa
    oc_ref[...] = sc

  @pl.when(pid != 0)
  def _():
    od_ref[...] += sd
    oa_ref[...] += sa
    oc_ref[...] += sc


def _pass1(xd, xa, xc, wd1, wa1, wc1):
  r = xd.shape[0]
  grid = (r // _TILE,)
  full = lambda s: pl.BlockSpec(s, lambda i: (0,) * len(s))
  st = jax.ShapeDtypeStruct((2, wd1.shape[1]), jnp.float32)
  return pl.pallas_call(
      _p1_body,
      grid=grid,
      in_specs=[
          pl.BlockSpec((_TILE, 64), lambda i: (i, 0)),
          pl.BlockSpec((_TILE, 64), lambda i: (i, 0)),
          pl.BlockSpec((_TILE, 1), lambda i: (i, 0)),
          full(wd1.shape), full(wa1.shape), full(wc1.shape),
      ],
      out_specs=[full((2, wd1.shape[1]))] * 3,
      out_shape=[st, st, st],
      compiler_params=pltpu.CompilerParams(
          dimension_semantics=("arbitrary",)),
  )(xd, xa, xc, wd1, wa1, wc1)


def _a1(x, w1, al, be):
  h1 = jnp.dot(x, w1, preferred_element_type=jnp.float32)
  return jnp.maximum(h1 * al + be, 0.0)


def _p2_body(m, xd_ref, xa_ref, xc_ref, s1d_ref, s1a_ref, s1c_ref,
             wd1_ref, gd1_ref, bd1_ref, wd2_ref,
             wa1_ref, ga1_ref, ba1_ref, wa2_ref,
             wc1_ref, gc1_ref, bc1_ref, wc2_ref,
             od_ref, oa_ref, oc_ref):
  pid = pl.program_id(0)

  def branch(x_ref, s1_ref, w1_ref, g1_ref, b1_ref, w2_ref, is_cd):
    al, be = _affine(s1_ref[...], g1_ref[...], b1_ref[...], m)
    if is_cd:
      a1 = jnp.maximum(x_ref[...] * w1_ref[...] * al + be, 0.0)
    else:
      a1 = _a1(x_ref[...], w1_ref[...], al, be)
    h2 = jnp.dot(a1, w2_ref[...], preferred_element_type=jnp.float32)
    return _stq(h2)

  sd = branch(xd_ref, s1d_ref, wd1_ref, gd1_ref, bd1_ref, wd2_ref, False)
  sa = branch(xa_ref, s1a_ref, wa1_ref, ga1_ref, ba1_ref, wa2_ref, False)
  sc = branch(xc_ref, s1c_ref, wc1_ref, gc1_ref, bc1_ref, wc2_ref, True)

  @pl.when(pid == 0)
  def _():
    od_ref[...] = sd
    oa_ref[...] = sa
    oc_ref[...] = sc

  @pl.when(pid != 0)
  def _():
    od_ref[...] += sd
    oa_ref[...] += sa
    oc_ref[...] += sc


def _pass2(xd, xa, xc, s1d, s1a, s1c,
           wd1, gd1, bd1, wd2, wa1, ga1, ba1, wa2, wc1, gc1, bc1, wc2):
  r = xd.shape[0]
  h = wd1.shape[1]
  grid = (r // _TILE,)
  full = lambda s: pl.BlockSpec(s, lambda i: (0,) * len(s))
  st = jax.ShapeDtypeStruct((2, h), jnp.float32)
  smalls = [s1d, s1a, s1c, wd1, gd1, bd1, wd2, wa1, ga1, ba1, wa2,
            wc1, gc1, bc1, wc2]
  return pl.pallas_call(
      functools.partial(_p2_body, float(r)),
      grid=grid,
      in_specs=[
          pl.BlockSpec((_TILE, 64), lambda i: (i, 0)),
          pl.BlockSpec((_TILE, 64), lambda i: (i, 0)),
          pl.BlockSpec((_TILE, 1), lambda i: (i, 0)),
      ] + [full(x.shape) for x in smalls],
      out_specs=[full((2, h))] * 3,
      out_shape=[st, st, st],
      compiler_params=pltpu.CompilerParams(
          dimension_semantics=("arbitrary",)),
  )(xd, xa, xc, *smalls)


def _p3_body(m, n, nh,
             xd_ref, xa_ref, xc_ref, sp_ref, spat_ref, ain_ref,
             acol_ref, arow_ref,
             s1d_ref, s2d_ref, s1a_ref, s2a_ref, s1c_ref, s2c_ref,
             wd1_ref, gd1_ref, bd1_ref, wd2_ref, gd2_ref, bd2_ref,
             wa1_ref, ga1_ref, ba1_ref, wa2_ref, ga2_ref, ba2_ref,
             wc1_ref, gc1_ref, bc1_ref, wc2_ref, gc2_ref, bc2_ref,
             tvd_ref, gam_ref, out_ref, carry_ref):
  rt = pl.program_id(1)

  def branch(x_ref, s1_ref, s2_ref, w1_ref, g1_ref, b1_ref,
             w2_ref, g2_ref, b2_ref, is_cd):
    al1, be1 = _affine(s1_ref[...], g1_ref[...], b1_ref[...], m)
    al2, be2 = _affine(s2_ref[...], g2_ref[...], b2_ref[...], m)
    if is_cd:
      a1 = jnp.maximum(x_ref[...] * w1_ref[...] * al1 + be1, 0.0)
    else:
      a1 = _a1(x_ref[...], w1_ref[...], al1, be1)
    h2 = jnp.dot(a1, w2_ref[...], preferred_element_type=jnp.float32)
    return jnp.maximum(h2 * al2 + be2, 0.0)

  s = branch(xd_ref, s1d_ref, s2d_ref, wd1_ref, gd1_ref, bd1_ref,
             wd2_ref, gd2_ref, bd2_ref, False)
  s += branch(xa_ref, s1a_ref, s2a_ref, wa1_ref, ga1_ref, ba1_ref,
              wa2_ref, ga2_ref, ba2_ref, False)
  s += branch(xc_ref, s1c_ref, s2c_ref, wc1_ref, gc1_ref, bc1_ref,
              wc2_ref, gc2_ref, bc2_ref, True)
  s += sp_ref[...]

  st = jnp.transpose(s, (1, 0)).reshape(nh, _TR, n)
  mask = (spat_ref[0] > 1)[None, :, :]
  ain = ain_ref[0][None, :, :]
  tot = ain + st
  gam = gam_ref[0, 0]
  inner = ain + jnp.where(mask, tot * gam, tot)

  tail = carry_ref[...]
  carry_ref[...] = inner[:, _TR - 1, :]
  rows = jnp.concatenate([tail[:, None, :], inner[:, :_TR - 1, :]], axis=1)

  tvec = tvd_ref[0, :]
  acol = acol_ref[0, pl.ds(rt * _TR, _TR), 0]
  border = 2.0 * acol[None, :] + tvec[:, None]
  chunk = jnp.concatenate([border[:, :, None], rows], axis=2)
  out_ref[0, :, pl.ds(rt * _TR, _TR), :] = chunk

  @pl.when(rt == 0)
  def _():
    row0 = arow_ref[0, 0, :]
    out_ref[0, :, 0, :] = 2.0 * row0[None, :] + tvec[:, None]

  @pl.when(rt == pl.num_programs(1) - 1)
  def _():
    last_col = 2.0 * acol_ref[0, n, 0] + tvec[:, None]
    out_ref[0, :, n, :] = jnp.concatenate(
        [last_col, inner[:, _TR - 1, :]], axis=1)


def _pass3(xd, xa, xc, sp_rows, spat, a_in, a_col, a_row0,
           stats, weights, tvd, gam):
  b, n, _ = spat.shape
  nh = tvd.shape[1]
  r = xd.shape[0]
  nt = n // _TR
  full = lambda s: pl.BlockSpec(s, lambda i, j: (0,) * len(s))
  rows = lambda w: pl.BlockSpec((_TILE, w), lambda i, j: (i * nt + j, 0))
  smalls = list(stats) + list(weights) + [tvd, gam]
  return pl.pallas_call(
      functools.partial(_p3_body, float(r), n, nh),
      grid=(b, nt),
      in_specs=[
          rows(64), rows(64), rows(1), rows(nh),
          pl.BlockSpec((1, _TR, n), lambda i, j: (i, j, 0)),
          pl.BlockSpec((1, _TR, n), lambda i, j: (i, j, 0)),
          pl.BlockSpec((1, n + 1, 1), lambda i, j: (i, 0, 0)),
          pl.BlockSpec((1, 1, n + 1), lambda i, j: (i, 0, 0)),
      ] + [full(x.shape) for x in smalls],
      out_specs=pl.BlockSpec((1, nh, n + 1, n + 1),
                             lambda i, j: (i, 0, 0, 0)),
      out_shape=jax.ShapeDtypeStruct((b, nh, n + 1, n + 1), jnp.float32),
      scratch_shapes=[pltpu.VMEM((nh, n), jnp.float32)],
      compiler_params=pltpu.CompilerParams(
          dimension_semantics=("arbitrary", "arbitrary")),
  )(xd, xa, xc, sp_rows, spat, a_in, a_col, a_row0, *smalls)


def kernel(attn_bias, spatial_pos, d2_distance, ang_distance,
           centroid_distance, edge_data, edge_type, edge_len, edge_ang,
           edge_conv, edge_path, edge_padding_mask, graph, node_feat,
           spatial_pos_W, gt_vd, gamma, d2_W1, d2_g1, d2_b1, d2_W2,
           d2_g2, d2_b2, ang_W1, ang_g1, ang_b1, ang_W2, ang_g2, ang_b2,
           cd_W1, cd_g1, cd_b1, cd_W2, cd_g2, cd_b2):
  b, n, _ = spatial_pos.shape
  h = spatial_pos_W.shape[1]
  r = b * n * n

  xd = d2_distance.reshape(r, 64)
  xa = ang_distance.reshape(r, 64)
  xc = centroid_distance.reshape(r, 1)
  idx = spatial_pos.reshape(r).astype(jnp.int32)

  sp_rows = _sc_gather(spatial_pos_W, idx)

  rs = lambda v: v.reshape(1, h)
  s1d, s1a, s1c = _pass1(xd, xa, xc, d2_W1, ang_W1, cd_W1)
  s2d, s2a, s2c = _pass2(
      xd, xa, xc, s1d, s1a, s1c,
      d2_W1, rs(d2_g1), rs(d2_b1), d2_W2,
      ang_W1, rs(ang_g1), rs(ang_b1), ang_W2,
      cd_W1, rs(cd_g1), rs(cd_b1), cd_W2)

  a_in = attn_bias[:, 1:, 1:]
  a_col = attn_bias[:, :, 0].reshape(b, n + 1, 1)
  a_row0 = attn_bias[:, 0, :].reshape(b, 1, n + 1)

  stats = (s1d, s2d, s1a, s2a, s1c, s2c)
  weights = (d2_W1, rs(d2_g1), rs(d2_b1), d2_W2, rs(d2_g2), rs(d2_b2),
             ang_W1, rs(ang_g1), rs(ang_b1), ang_W2, rs(ang_g2), rs(ang_b2),
             cd_W1, rs(cd_g1), rs(cd_b1), cd_W2, rs(cd_g2), rs(cd_b2))
  return _pass3(xd, xa, xc, sp_rows, spatial_pos, a_in, a_col, a_row0,
                stats, weights, gt_vd, gamma.reshape(1, 1))
